# baseline (speedup 1.0000x reference)
import jax
import jax.numpy as jnp
from jax.experimental import pallas as pl

_ROWS = 320000
_COLS = 128
_BLOCK_ROWS = 40000


def _sum_block_kernel(x_ref, o_ref):
    i = pl.program_id(0)

    @pl.when(i == 0)
    def _init():
        o_ref[...] = jnp.zeros_like(o_ref)

    o_ref[...] += jnp.sum(x_ref[...], axis=0, keepdims=True)


def kernel(x):
    grid = _ROWS // _BLOCK_ROWS
    out = pl.pallas_call(
        _sum_block_kernel,
        grid=(grid,),
        in_specs=[pl.BlockSpec((_BLOCK_ROWS, _COLS), lambda i: (i, 0))],
        out_specs=pl.BlockSpec((1, _COLS), lambda i: (0, 0)),
        out_shape=jax.ShapeDtypeStruct((1, _COLS), jnp.float32),
    )(x)
    return out
